# baseline (speedup 1.0000x reference)
import functools

import jax
import jax.numpy as jnp
from jax import lax
from jax.experimental import pallas as pl
from jax.experimental.pallas import tpu as pltpu
from jax.experimental.pallas import tpu_sc as plsc

B, V, K = 128, 100000, 64
NW = 32
ROWS_PER_W = B // NW
CHUNK = 20000
NCHUNK = V // CHUNK


def _sc_mask_kernel(scores_hbm, allowed_hbm, out_hbm,
                    negbuf, idx0, idx1, idx2, idx3,
                    val0, val1, val2, val3, gsem, fsem, ssem):
    c = lax.axis_index("c")
    s = lax.axis_index("s")
    wid = s * 2 + c
    idx_bufs = (idx0, idx1, idx2, idx3)
    val_bufs = (val0, val1, val2, val3)

    neg = jnp.full((16,), -jnp.inf, dtype=jnp.float32)

    def fill(i, carry):
        negbuf[pl.ds(i * 16, 16)] = neg
        return carry

    lax.fori_loop(0, CHUNK // 16, fill, 0)

    gathers = []
    for r in range(ROWS_PER_W):
        row = wid * ROWS_PER_W + r
        pltpu.sync_copy(allowed_hbm.at[row], idx_bufs[r])
        base = row * V
        for q in range(K // 16):
            sl = pl.ds(q * 16, 16)
            idx_bufs[r][sl] = idx_bufs[r][sl] + base
        gathers.append(pltpu.async_copy(scores_hbm.at[idx_bufs[r]],
                                        val_bufs[r], gsem))

    fills = []
    for r in range(ROWS_PER_W):
        row = wid * ROWS_PER_W + r
        for j in range(NCHUNK):
            dst = out_hbm.at[pl.ds(row * V + j * CHUNK, CHUNK)]
            fills.append(pltpu.async_copy(negbuf, dst, fsem))
    for h in fills:
        h.wait()
    for h in gathers:
        h.wait()

    scats = []
    for r in range(ROWS_PER_W):
        scats.append(pltpu.async_copy(val_bufs[r],
                                      out_hbm.at[idx_bufs[r]], ssem))
    for h in scats:
        h.wait()


@jax.jit
def _masked_scores(scores, allowed_ids):
    mesh = plsc.VectorSubcoreMesh(core_axis_name="c", subcore_axis_name="s")
    run = functools.partial(
        pl.kernel,
        out_type=jax.ShapeDtypeStruct((B * V,), jnp.float32),
        mesh=mesh,
        scratch_types=[
            pltpu.VMEM((CHUNK,), jnp.float32),
            pltpu.VMEM((K,), jnp.int32),
            pltpu.VMEM((K,), jnp.int32),
            pltpu.VMEM((K,), jnp.int32),
            pltpu.VMEM((K,), jnp.int32),
            pltpu.VMEM((K,), jnp.float32),
            pltpu.VMEM((K,), jnp.float32),
            pltpu.VMEM((K,), jnp.float32),
            pltpu.VMEM((K,), jnp.float32),
            pltpu.SemaphoreType.DMA,
            pltpu.SemaphoreType.DMA,
            pltpu.SemaphoreType.DMA,
        ],
    )(_sc_mask_kernel)
    flat = run(scores.reshape(-1), allowed_ids)
    return flat.reshape(B, V)


def kernel(input_ids, scores, allowed_ids):
    del input_ids
    return _masked_scores(scores, allowed_ids)


# speedup vs baseline: 1.6868x; 1.6868x over previous
import functools

import jax
import jax.numpy as jnp
from jax import lax
from jax.experimental import pallas as pl
from jax.experimental.pallas import tpu as pltpu
from jax.experimental.pallas import tpu_sc as plsc

B, V, K = 128, 100000, 64
NW = 32
ROWS_PER_W = B // NW
CHUNK = 20000
NCHUNK = V // CHUNK


def _sc_mask_kernel(scores_hbm, allowed_hbm, out_hbm,
                    negbuf, alw, slab, idxr, gsem, fsem):
    c = lax.axis_index("c")
    s = lax.axis_index("s")
    wid = s * 2 + c
    base_row = wid * ROWS_PER_W
    grp = (wid // 2) * 8

    pltpu.sync_copy(allowed_hbm.at[pl.ds(grp, 8)], alw)

    neg = jnp.full((16,), -jnp.inf, dtype=jnp.float32)

    def fill(i, carry):
        negbuf[pl.ds(i * 16, 16)] = neg
        return carry

    lax.fori_loop(0, V // 16, fill, 0)

    lane = lax.iota(jnp.int32, 16)
    for r in range(ROWS_PER_W):
        row = base_row + r
        lr = row - grp

        slabs = []
        for q in range(K // 16):
            id16 = alw[lr, pl.ds(q * 16, 16)]
            for j in range(16):
                idv = id16[j]
                g = pl.multiple_of((idv >> 7) * 128, 128)
                src = scores_hbm.at[row].at[pl.ds(g, 128)]
                slabs.append(pltpu.async_copy(src, slab.at[q * 16 + j], gsem))
        for h in slabs:
            h.wait()

        for q in range(K // 16):
            id16 = alw[lr, pl.ds(q * 16, 16)]
            idxr[pl.ds(q * 16, 16)] = id16
            k16 = lane + (q * 16)
            off16 = jnp.bitwise_and(id16, 127)
            v16 = plsc.load_gather(slab, [k16, off16])
            plsc.store_scatter(negbuf, [id16], v16)

        pltpu.async_copy(negbuf, out_hbm.at[row], fsem).wait()
        for q in range(K // 16):
            id16 = idxr[pl.ds(q * 16, 16)]
            plsc.store_scatter(negbuf, [id16], neg)


@jax.jit
def _masked_scores(scores, allowed_ids):
    mesh = plsc.VectorSubcoreMesh(core_axis_name="c", subcore_axis_name="s")
    run = functools.partial(
        pl.kernel,
        out_type=jax.ShapeDtypeStruct((B, V), jnp.float32),
        mesh=mesh,
        compiler_params=pltpu.CompilerParams(needs_layout_passes=False),
        scratch_types=[
            pltpu.VMEM((V,), jnp.float32),
            pltpu.VMEM((8, K), jnp.int32),
            pltpu.VMEM((K, 128), jnp.float32),
            pltpu.VMEM((K,), jnp.int32),
            pltpu.SemaphoreType.DMA,
            pltpu.SemaphoreType.DMA,
        ],
    )(_sc_mask_kernel)
    return run(scores, allowed_ids)


def kernel(input_ids, scores, allowed_ids):
    del input_ids
    return _masked_scores(scores, allowed_ids)
